# baseline (speedup 1.0000x reference)
import functools
import math

import jax
import jax.numpy as jnp
from jax import lax
from jax.experimental import pallas as pl
from jax.experimental.pallas import tpu as pltpu
from jax.experimental.pallas import tpu_sc as plsc

VOCAB = 1000000
D = 64
SCALE = math.sqrt(D)

NC = 2
NS = 16
NW = NC * NS

NB = 16384
NP = 20
BPW = NB // NW
HALF = BPW // 2
NSTEP = NP * 2
FT = D // 8
BT = NB // 128
BTW = BPW // 128


UCH = 80
NCHU = VOCAB // UCH
NIT = 392


@functools.partial(
    pl.kernel,
    mesh=plsc.VectorSubcoreMesh(core_axis_name="c", subcore_axis_name="s"),
    out_type=jax.ShapeDtypeStruct((VOCAB, 128), jnp.float32),
    scratch_types=[
        pltpu.VMEM((2, UCH, D), jnp.float32),
        pltpu.VMEM((2, UCH, 128), jnp.float32),
        pltpu.SemaphoreType.DMA((2,)),
        pltpu.SemaphoreType.DMA((2,)),
    ],
    compiler_params=pltpu.CompilerParams(
        use_tc_tiling_on_sc=True, needs_layout_passes=False
    ),
)
def _untile_kernel(tin, tout, buf, wide, isem, osem):
    cid = lax.axis_index("c")
    sid = lax.axis_index("s")
    wid = sid * NC + cid

    def start_in(c, b):
        pltpu.async_copy(tin.at[pl.ds(c * UCH, UCH)], buf.at[b], isem.at[b])

    def move_out(c, b):
        pltpu.make_async_copy(
            tin.at[pl.ds(0, UCH)], buf.at[b], isem.at[b]
        ).wait()

        def widen(r, carry):
            for j in range(D // 16):
                sl = pl.ds(j * 16, 16)
                wide[b, r, sl] = buf[b, r, sl] * SCALE
            return carry

        lax.fori_loop(0, UCH, widen, 0)
        pltpu.async_copy(
            wide.at[b], tout.at[pl.ds(c * UCH, UCH)], osem.at[b]
        )
        pltpu.make_async_copy(
            wide.at[b], tout.at[pl.ds(0, UCH)], osem.at[b]
        ).wait()

    start_in(wid, 0)

    @pl.loop(0, NIT, step=2)
    def _i(i):
        for b in range(2):
            c = wid + (i + b) * NW
            nxt = c + NW

            @pl.when(nxt < NCHU)
            def _s():
                start_in(nxt, 1 - b)

            @pl.when(c < NCHU)
            def _m():
                move_out(c, b)


@functools.partial(
    pl.kernel,
    mesh=plsc.VectorSubcoreMesh(core_axis_name="c", subcore_axis_name="s"),
    out_type=jax.ShapeDtypeStruct((NP, FT, BT, 8, 128), jnp.float32),
    scratch_types=[
        pltpu.VMEM((NP, BPW), jnp.int32),
        pltpu.VMEM((2, HALF, 128), jnp.float32),
        pltpu.VMEM((2, FT, 2, 8, 129), jnp.float32),
        pltpu.SemaphoreType.DMA((2,)),
        pltpu.SemaphoreType.DMA((2,)),
    ],
    compiler_params=pltpu.CompilerParams(
        use_tc_tiling_on_sc=False, needs_layout_passes=False
    ),
)
def _embed_kernel(xt_hbm, table_hbm, out_hbm, idx_v, gbuf, tbuf, gsem, ssem):
    cid = lax.axis_index("c")
    sid = lax.axis_index("s")
    wid = sid * NC + cid
    b0 = wid * BPW

    for p in range(NP):
        pltpu.sync_copy(xt_hbm.at[p, pl.ds(b0, BPW)], idx_v.at[p])

    iota = lax.iota(jnp.int32, 16)

    def gathers(s, b, start):
        p = s // 2
        h = s % 2
        for jj in range(2):
            idxref = idx_v.at[p, pl.ds(h * HALF + jj * 128, 128)]
            cp = (pltpu.async_copy if start else pltpu.make_async_copy)(
                table_hbm.at[idxref],
                gbuf.at[b, pl.ds(jj * 128, 128)],
                gsem.at[b],
            )
            if not start:
                cp.wait()

    def scatters(s, b, start):
        p = s // 2
        h = s % 2
        bt0 = wid * BTW + h * 2
        for ft in range(FT):
            for bl in range(2):
                cp = (pltpu.async_copy if start else pltpu.make_async_copy)(
                    tbuf.at[b, ft, bl, :, pl.ds(0, 128)],
                    out_hbm.at[p, ft, bt0 + bl],
                    ssem.at[b],
                )
                if not start:
                    cp.wait()

    zeros16 = jnp.full((16,), 0, jnp.int32)
    ftv = [(iota + j * 16) // 8 for j in range(4)]
    fiv = [(iota + j * 16) % 8 for j in range(4)]

    def transpose_scale(b):
        @plsc.parallel_loop(0, HALF, unroll=2)
        def _r(r):
            btv = zeros16 + r // 128
            biv = zeros16 + r % 128
            for j in range(4):
                vals = gbuf[b, r, pl.ds(j * 16, 16)]
                plsc.store_scatter(tbuf.at[b], [ftv[j], btv, fiv[j], biv], vals)

    def step(s, b, first, last):
        gathers(s, b, start=False)
        if not last:
            gathers(s + 1, 1 - b, start=True)
        if not first:
            scatters(s - 2, b, start=False)
        transpose_scale(b)
        scatters(s, b, start=True)

    gathers(0, 0, start=True)
    step(0, 0, first=True, last=False)
    step(1, 1, first=True, last=False)

    @pl.loop(2, NSTEP - 2, step=2)
    def _steady(s):
        step(s, 0, first=False, last=False)
        step(s + 1, 1, first=False, last=False)

    step(NSTEP - 2, 0, first=False, last=False)
    step(NSTEP - 1, 1, first=False, last=True)
    scatters(NSTEP - 2, 0, start=False)
    scatters(NSTEP - 1, 1, start=False)


def kernel(x, table):
    xt = jnp.transpose(x.astype(jnp.int32))
    out5d = _embed_kernel(xt, _untile_kernel(table))
    return out5d.transpose(2, 4, 0, 1, 3).reshape(NB, NP, D)


# speedup vs baseline: 1.2888x; 1.2888x over previous
import functools
import math

import jax
import jax.numpy as jnp
from jax import lax
from jax.experimental import pallas as pl
from jax.experimental.pallas import tpu as pltpu
from jax.experimental.pallas import tpu_sc as plsc

VOCAB = 1000000
D = 64
SCALE = math.sqrt(D)

NC = 2
NS = 16
NW = NC * NS

NB = 16384
NP = 20
BPW = NB // NW
HALF = BPW // 2
NSTEP = NP * 2
FT = D // 8
BT = NB // 128
BTW = BPW // 128


@functools.partial(
    pl.kernel,
    mesh=plsc.VectorSubcoreMesh(core_axis_name="c", subcore_axis_name="s"),
    out_type=jax.ShapeDtypeStruct((NP, FT, BT, 8, 128), jnp.float32),
    scratch_types=[
        pltpu.VMEM((NP, BPW), jnp.int32),
        pltpu.VMEM((2, HALF, D), jnp.float32),
        pltpu.VMEM((2, FT, 2, 8, 129), jnp.float32),
        pltpu.SemaphoreType.DMA((2,)),
        pltpu.SemaphoreType.DMA((2,)),
    ],
    compiler_params=pltpu.CompilerParams(
        use_tc_tiling_on_sc=False, needs_layout_passes=False
    ),
)
def _embed_kernel(xt_hbm, table_hbm, out_hbm, idx_v, gbuf, tbuf, gsem, ssem):
    cid = lax.axis_index("c")
    sid = lax.axis_index("s")
    wid = sid * NC + cid
    b0 = wid * BPW

    for p in range(NP):
        pltpu.sync_copy(xt_hbm.at[p, pl.ds(b0, BPW)], idx_v.at[p])

    iota = lax.iota(jnp.int32, 16)

    def gathers(s, b, start):
        p = s // 2
        h = s % 2
        for jj in range(2):
            idxref = idx_v.at[p, pl.ds(h * HALF + jj * 128, 128)]
            cp = (pltpu.async_copy if start else pltpu.make_async_copy)(
                table_hbm.at[idxref],
                gbuf.at[b, pl.ds(jj * 128, 128)],
                gsem.at[b],
            )
            if not start:
                cp.wait()

    def scatters(s, b, start):
        p = s // 2
        h = s % 2
        bt0 = wid * BTW + h * 2
        for ft in range(FT):
            for bl in range(2):
                cp = (pltpu.async_copy if start else pltpu.make_async_copy)(
                    tbuf.at[b, ft, bl, :, pl.ds(0, 128)],
                    out_hbm.at[p, ft, bt0 + bl],
                    ssem.at[b],
                )
                if not start:
                    cp.wait()

    zeros16 = jnp.full((16,), 0, jnp.int32)
    ftv = [(iota + j * 16) // 8 for j in range(4)]
    fiv = [(iota + j * 16) % 8 for j in range(4)]

    def transpose_scale(b):
        @plsc.parallel_loop(0, HALF, unroll=2)
        def _r(r):
            btv = zeros16 + r // 128
            biv = zeros16 + r % 128
            for j in range(4):
                vals = gbuf[b, r, pl.ds(j * 16, 16)] * SCALE
                plsc.store_scatter(tbuf.at[b], [ftv[j], btv, fiv[j], biv], vals)

    def step(s, b, first, last):
        gathers(s, b, start=False)
        if not last:
            gathers(s + 1, 1 - b, start=True)
        if not first:
            scatters(s - 2, b, start=False)
        transpose_scale(b)
        scatters(s, b, start=True)

    gathers(0, 0, start=True)
    step(0, 0, first=True, last=False)
    step(1, 1, first=True, last=False)

    @pl.loop(2, NSTEP - 2, step=2)
    def _steady(s):
        step(s, 0, first=False, last=False)
        step(s + 1, 1, first=False, last=False)

    step(NSTEP - 2, 0, first=False, last=False)
    step(NSTEP - 1, 1, first=False, last=True)
    scatters(NSTEP - 2, 0, start=False)
    scatters(NSTEP - 1, 1, start=False)


def kernel(x, table):
    xt = jnp.transpose(x.astype(jnp.int32))
    out5d = _embed_kernel(xt, table)
    return out5d.transpose(2, 4, 0, 1, 3).reshape(NB, NP, D)


# speedup vs baseline: 1.2899x; 1.0008x over previous
import functools
import math

import jax
import jax.numpy as jnp
from jax import lax
from jax.experimental import pallas as pl
from jax.experimental.pallas import tpu as pltpu
from jax.experimental.pallas import tpu_sc as plsc

VOCAB = 1000000
D = 64
SCALE = math.sqrt(D)

NC = 2
NS = 16
NW = NC * NS

NB = 16384
NP = 20
BPW = NB // NW
HALF = BPW // 2
NSTEP = NP * 2
FT = D // 8
BT = NB // 128
BTW = BPW // 128


@functools.partial(
    pl.kernel,
    mesh=plsc.VectorSubcoreMesh(core_axis_name="c", subcore_axis_name="s"),
    out_type=jax.ShapeDtypeStruct((NP, FT, BT, 8, 128), jnp.float32),
    scratch_types=[
        pltpu.VMEM((NP, BPW), jnp.int32),
        pltpu.VMEM((2, HALF, D), jnp.float32),
        pltpu.VMEM((2, FT, 2, 8, 129), jnp.float32),
        pltpu.SemaphoreType.DMA((2,)),
        pltpu.SemaphoreType.DMA((2,)),
    ],
    compiler_params=pltpu.CompilerParams(
        use_tc_tiling_on_sc=False, needs_layout_passes=False
    ),
)
def _embed_kernel(xt_hbm, table_hbm, out_hbm, idx_v, gbuf, tbuf, gsem, ssem):
    cid = lax.axis_index("c")
    sid = lax.axis_index("s")
    wid = sid * NC + cid
    b0 = wid * BPW

    for p in range(NP):
        pltpu.sync_copy(xt_hbm.at[p, pl.ds(b0, BPW)], idx_v.at[p])

    iota = lax.iota(jnp.int32, 16)

    def gathers(s, b, start):
        p = s // 2
        h = s % 2
        for jj in range(2):
            idxref = idx_v.at[p, pl.ds(h * HALF + jj * 128, 128)]
            cp = (pltpu.async_copy if start else pltpu.make_async_copy)(
                table_hbm.at[idxref],
                gbuf.at[b, pl.ds(jj * 128, 128)],
                gsem.at[b],
            )
            if not start:
                cp.wait()

    def scatters(s, b, start):
        p = s // 2
        h = s % 2
        bt0 = wid * BTW + h * 2
        for ft in range(FT):
            for bl in range(2):
                cp = (pltpu.async_copy if start else pltpu.make_async_copy)(
                    tbuf.at[b, ft, bl, :, pl.ds(0, 128)],
                    out_hbm.at[p, ft, bt0 + bl],
                    ssem.at[b],
                )
                if not start:
                    cp.wait()

    zeros16 = jnp.full((16,), 0, jnp.int32)
    ftv = [(iota + j * 16) // 8 for j in range(4)]
    fiv = [(iota + j * 16) % 8 for j in range(4)]

    def transpose_scale(b):
        @plsc.parallel_loop(0, HALF, unroll=4)
        def _r(r):
            btv = zeros16 + r // 128
            biv = zeros16 + r % 128
            for j in range(4):
                vals = gbuf[b, r, pl.ds(j * 16, 16)] * SCALE
                plsc.store_scatter(tbuf.at[b], [ftv[j], btv, fiv[j], biv], vals)

    def step(s, b, first, last):
        gathers(s, b, start=False)
        if not last:
            gathers(s + 1, 1 - b, start=True)
        if not first:
            scatters(s - 2, b, start=False)
        transpose_scale(b)
        scatters(s, b, start=True)

    gathers(0, 0, start=True)
    step(0, 0, first=True, last=False)
    step(1, 1, first=True, last=False)

    @pl.loop(2, NSTEP - 2, step=2)
    def _steady(s):
        step(s, 0, first=False, last=False)
        step(s + 1, 1, first=False, last=False)

    step(NSTEP - 2, 0, first=False, last=False)
    step(NSTEP - 1, 1, first=False, last=True)
    scatters(NSTEP - 2, 0, start=False)
    scatters(NSTEP - 1, 1, start=False)


def kernel(x, table):
    xt = jnp.transpose(x.astype(jnp.int32))
    out5d = _embed_kernel(xt, table)
    return out5d.transpose(2, 4, 0, 1, 3).reshape(NB, NP, D)
